# baseline (speedup 1.0000x reference)
import functools

import jax
import jax.numpy as jnp
from jax import lax
from jax.experimental import pallas as pl
from jax.experimental.pallas import tpu as pltpu
from jax.experimental.pallas import tpu_sc as plsc

NN = 10000
NE = 160000
D = 128
H = 8
CQ = 16
CK = 16
TEMP = 4.0

EBLK = 2000
NBLK = 2000

SC_NC = 2
SC_NS = 16
SC_NW = SC_NC * SC_NS


def _bdot(a, b):
    return jnp.dot(a.astype(jnp.bfloat16), b.astype(jnp.bfloat16),
                   preferred_element_type=jnp.float32)


def _gru0_body(x_ref, wih_ref, bih_ref, bhh_ref, out_ref):
    gi = _bdot(x_ref[...], wih_ref[...]) + bih_ref[...]
    gh = bhh_ref[...]
    r = jax.nn.sigmoid(gi[:, :D] + gh[:, :D])
    z = jax.nn.sigmoid(gi[:, D:2 * D] + gh[:, D:2 * D])
    n = jnp.tanh(gi[:, 2 * D:] + r * gh[:, 2 * D:])
    out_ref[...] = (1.0 - z) * n


def _gru0(x, wih_t, bih, bhh):
    nb = x.shape[0] // NBLK
    return pl.pallas_call(
        _gru0_body,
        grid=(nb,),
        in_specs=[
            pl.BlockSpec((NBLK, D), lambda i: (i, 0)),
            pl.BlockSpec((D, 3 * D), lambda i: (0, 0)),
            pl.BlockSpec((1, 3 * D), lambda i: (0, 0)),
            pl.BlockSpec((1, 3 * D), lambda i: (0, 0)),
        ],
        out_specs=pl.BlockSpec((NBLK, D), lambda i: (i, 0)),
        out_shape=jax.ShapeDtypeStruct((x.shape[0], D), jnp.float32),
    )(x, wih_t, bih, bhh)


def _edge_body(first,
               ea_ref, xi_ref, xj_ref,
               wih_ref, bih_ref, whh_ref, bhh_ref,
               w1a_ref, w1b_ref, w1c_ref, be1_ref, w2_ref, be2_ref,
               wq_ref, bq_ref, wk_ref, bk_ref, wv_ref, bv_ref,
               m1_ref, a1_ref, m2_ref, a2_ref, g_ref, gt_ref,
               enext_ref, prob_ref, value_ref):
    f32 = jnp.float32
    ea = ea_ref[...]
    xi = xi_ref[...]
    xj = xj_ref[...]

    if first:
        gi = _bdot(ea, wih_ref[...]) + bih_ref[...]
        gh = bhh_ref[...]
        r = jax.nn.sigmoid(gi[:, :D] + gh[:, :D])
        z = jax.nn.sigmoid(gi[:, D:2 * D] + gh[:, D:2 * D])
        n = jnp.tanh(gi[:, 2 * D:] + r * gh[:, 2 * D:])
        e = (1.0 - z) * n
    else:
        e = ea

    h1 = jax.nn.relu(
        _bdot(xi, w1a_ref[...])
        + _bdot(e, w1b_ref[...])
        + _bdot(xj, w1c_ref[...])
        + be1_ref[...])
    emsg = _bdot(h1, w2_ref[...]) + be2_ref[...]

    q = _bdot(xi, wq_ref[...]) + bq_ref[...]
    k = _bdot(e, wk_ref[...]) + bk_ref[...]
    v = _bdot(xj, wv_ref[...]) + bv_ref[...]
    ain = jnp.concatenate([q, k], axis=1)
    hcn = jax.nn.relu(
        _bdot(ain, m1_ref[...]) + a1_ref[...])
    att = (_bdot(hcn, m2_ref[...])
           + a2_ref[...]) * (1.0 / TEMP)

    m = jnp.max(att, axis=1, keepdims=True)
    ex = jnp.exp(att - m)
    ssum = _bdot(ex, g_ref[...])
    denom = _bdot(ssum, gt_ref[...])
    prob = ex / denom
    prob_ref[...] = prob
    value_ref[...] = prob * v

    msg = jax.nn.relu(emsg) if first else emsg
    gi2 = _bdot(msg, wih_ref[...]) + bih_ref[...]
    gh2 = _bdot(e, whh_ref[...]) + bhh_ref[...]
    r2 = jax.nn.sigmoid(gi2[:, :D] + gh2[:, :D])
    z2 = jax.nn.sigmoid(gi2[:, D:2 * D] + gh2[:, D:2 * D])
    n2 = jnp.tanh(gi2[:, 2 * D:] + r2 * gh2[:, 2 * D:])
    enext_ref[...] = (1.0 - z2) * n2 + z2 * e


def _edge_layer(first, ea, xi, xj, ge, lp):
    nb = NE // EBLK
    blk = lambda r, c: pl.BlockSpec((r, c), lambda i: (0, 0))
    outs = pl.pallas_call(
        functools.partial(_edge_body, first),
        grid=(nb,),
        in_specs=[
            pl.BlockSpec((EBLK, D), lambda i: (i, 0)),
            pl.BlockSpec((EBLK, D), lambda i: (i, 0)),
            pl.BlockSpec((EBLK, D), lambda i: (i, 0)),
            blk(D, 3 * D), blk(1, 3 * D), blk(D, 3 * D), blk(1, 3 * D),
            blk(D, 2 * D), blk(D, 2 * D), blk(D, 2 * D), blk(1, 2 * D),
            blk(2 * D, D), blk(1, D),
            blk(D, D), blk(1, D), blk(D, D), blk(1, D), blk(D, D), blk(1, D),
            blk(2 * D, 2 * D), blk(1, 2 * D), blk(2 * D, D), blk(1, D),
            blk(D, H), blk(H, D),
        ],
        out_specs=[
            pl.BlockSpec((EBLK, D), lambda i: (i, 0)),
            pl.BlockSpec((EBLK, D), lambda i: (i, 0)),
            pl.BlockSpec((EBLK, D), lambda i: (i, 0)),
        ],
        out_shape=[
            jax.ShapeDtypeStruct((NE, D), jnp.float32),
            jax.ShapeDtypeStruct((NE, D), jnp.float32),
            jax.ShapeDtypeStruct((NE, D), jnp.float32),
        ],
    )(ea, xi, xj,
      ge['wih'], ge['bih'], ge['whh'], ge['bhh'],
      lp['w1a'], lp['w1b'], lp['w1c'], lp['be1'], lp['w2'], lp['be2'],
      lp['wq'], lp['bq'], lp['wk'], lp['bk'], lp['wv'], lp['bv'],
      lp['m1'], lp['a1'], lp['m2'], lp['a2'], lp['g'], lp['gt'])
    return outs


def _node_body(first,
               node_ref, agga_ref, aggb_ref,
               u1a_ref, u1b_ref, u1_ref, u2_ref, u2b_ref,
               wih_ref, bih_ref, whh_ref, bhh_ref,
               out_ref):
    f32 = jnp.float32
    node = node_ref[...]
    agg = agga_ref[...] + aggb_ref[...]
    u = jax.nn.relu(
        _bdot(node, u1a_ref[...])
        + _bdot(agg, u1b_ref[...])
        + u1_ref[...])
    nmsg = _bdot(u, u2_ref[...]) + u2b_ref[...]
    if first:
        nmsg = jax.nn.relu(nmsg)
    gi = _bdot(nmsg, wih_ref[...]) + bih_ref[...]
    gh = _bdot(node, whh_ref[...]) + bhh_ref[...]
    r = jax.nn.sigmoid(gi[:, :D] + gh[:, :D])
    z = jax.nn.sigmoid(gi[:, D:2 * D] + gh[:, D:2 * D])
    n = jnp.tanh(gi[:, 2 * D:] + r * gh[:, 2 * D:])
    out_ref[...] = (1.0 - z) * n + z * node


def _node_layer(first, node, agg_a, agg_b, gn, lp):
    nb = NN // NBLK
    blk = lambda r, c: pl.BlockSpec((r, c), lambda i: (0, 0))
    return pl.pallas_call(
        functools.partial(_node_body, first),
        grid=(nb,),
        in_specs=[
            pl.BlockSpec((NBLK, D), lambda i: (i, 0)),
            pl.BlockSpec((NBLK, D), lambda i: (i, 0)),
            pl.BlockSpec((NBLK, D), lambda i: (i, 0)),
            blk(D, 2 * D), blk(D, 2 * D), blk(1, 2 * D),
            blk(2 * D, D), blk(1, D),
            blk(D, 3 * D), blk(1, 3 * D), blk(D, 3 * D), blk(1, 3 * D),
        ],
        out_specs=pl.BlockSpec((NBLK, D), lambda i: (i, 0)),
        out_shape=jax.ShapeDtypeStruct((NN, D), jnp.float32),
    )(node, agg_a, agg_b,
      lp['u1a'], lp['u1b'], lp['u1'], lp['u2'], lp['u2b'],
      gn['wih'], gn['bih'], gn['whh'], gn['bhh'])


G_ROWS = 2 * NE
G_PER_W = G_ROWS // SC_NW
G_CHUNK = 400
G_ITERS = G_PER_W // G_CHUNK


def _gather_body(table, idx, out, idx_v, rows_v, sem):
    c = lax.axis_index("c")
    s = lax.axis_index("s")
    wid = s * SC_NC + c
    base = wid * G_PER_W
    for i in range(G_ITERS):
        off = base + i * G_CHUNK
        pltpu.sync_copy(idx.at[pl.ds(off, G_CHUNK)], idx_v)
        pltpu.async_copy(table.at[idx_v], rows_v, sem).wait()
        pltpu.sync_copy(rows_v, out.at[pl.ds(off, G_CHUNK)])


def _sc_gather(table, idx_cat):
    call = pl.kernel(
        _gather_body,
        out_type=jax.ShapeDtypeStruct((G_ROWS, D), jnp.float32),
        mesh=plsc.VectorSubcoreMesh(core_axis_name="c", subcore_axis_name="s"),
        scratch_types=[
            pltpu.VMEM((G_CHUNK,), jnp.int32),
            pltpu.VMEM((G_CHUNK, D), jnp.float32),
            pltpu.SemaphoreType.DMA,
        ],
    )
    return call(table, idx_cat)


S_PER_SC = NE // SC_NC
S_PER_T = S_PER_SC // SC_NS
S_CHUNK = 200
S_ITERS = S_PER_T // S_CHUNK
NN_PAD = 10112
Z_ROWS = NN_PAD // SC_NS


def _scatter_body(value, idx, zrows, out, agg_sp, idx_v, val_v):
    c = lax.axis_index("c")
    s = lax.axis_index("s")
    pltpu.sync_copy(zrows, agg_sp.at[pl.ds(s * Z_ROWS, Z_ROWS)])
    plsc.subcore_barrier()
    base = c * S_PER_SC + s * S_PER_T
    for i in range(S_ITERS):
        off = base + i * S_CHUNK
        pltpu.sync_copy(idx.at[pl.ds(off, S_CHUNK)], idx_v)
        pltpu.sync_copy(value.at[pl.ds(off, S_CHUNK)], val_v)
        pltpu.sync_copy(val_v, agg_sp.at[idx_v], add=True)
    plsc.subcore_barrier()
    pltpu.sync_copy(agg_sp.at[pl.ds(s * Z_ROWS, Z_ROWS)],
                    out.at[c, pl.ds(s * Z_ROWS, Z_ROWS)])


def _sc_scatter(value, idx_i):
    call = pl.kernel(
        _scatter_body,
        out_type=jax.ShapeDtypeStruct((SC_NC, NN_PAD, D), jnp.float32),
        mesh=plsc.VectorSubcoreMesh(core_axis_name="c", subcore_axis_name="s"),
        scratch_types=[
            pltpu.VMEM_SHARED((NN_PAD, D), jnp.float32),
            pltpu.VMEM((S_CHUNK,), jnp.int32),
            pltpu.VMEM((S_CHUNK, D), jnp.float32),
        ],
    )
    zrows = jnp.zeros((Z_ROWS, D), jnp.float32)
    out = call(value, idx_i, zrows)
    return out[:, :NN]


def _prep_gru(p):
    return {
        'wih': p['Wih'].T,
        'whh': p['Whh'].T,
        'bih': p['bih'].reshape(1, -1),
        'bhh': p['bhh'].reshape(1, -1),
    }


def _prep_layer(p):
    eye = jnp.eye(H, dtype=jnp.float32)
    m1 = (p['A1'].T[:, None, :, None] * eye[None, :, None, :]).reshape(2 * D, 2 * D)
    m2 = (p['A2'].T[:, None, :, None] * eye[None, :, None, :]).reshape(2 * D, D)
    w1 = p['We1'].T
    g = jnp.tile(eye, (CK, 1))
    return {
        'w1a': w1[:D], 'w1b': w1[D:2 * D], 'w1c': w1[2 * D:],
        'be1': p['be1'].reshape(1, -1),
        'w2': p['We2'].T, 'be2': p['be2'].reshape(1, -1),
        'wq': p['Wq'].T, 'bq': p['bq'].reshape(1, -1),
        'wk': p['Wk'].T, 'bk': p['bk'].reshape(1, -1),
        'wv': p['Wv'].T, 'bv': p['bv'].reshape(1, -1),
        'm1': m1, 'a1': jnp.repeat(p['a1'], H).reshape(1, -1),
        'm2': m2, 'a2': jnp.repeat(p['a2'], H).reshape(1, -1),
        'g': g, 'gt': g.T,
        'u1a': p['U1'].T[:D], 'u1b': p['U1'].T[D:],
        'u1': p['u1'].reshape(1, -1),
        'u2': p['U2'].T, 'u2b': p['u2'].reshape(1, -1),
    }


def kernel(x, edge_attr, edge_index, params):
    idx_i = edge_index[0]
    idx_cat = jnp.concatenate([edge_index[0], edge_index[1]])
    gn = _prep_gru(params['gru_node'])
    ge = _prep_gru(params['gru_edge'])

    node = _gru0(x, gn['wih'], gn['bih'], gn['bhh'])
    edge = edge_attr
    probs = []
    for li in range(2):
        lp = _prep_layer(params['layers'][li])
        first = li == 0
        gath = _sc_gather(node, idx_cat)
        xi = gath[:NE]
        xj = gath[NE:]
        edge, prob, value = _edge_layer(first, edge, xi, xj, ge, lp)
        parts = _sc_scatter(value, idx_i)
        node = _node_layer(first, node, parts[0], parts[1], gn, lp)
        probs.append(prob.reshape(NE, CK, H))
    return node, edge, jnp.stack(probs)


# speedup vs baseline: 1.0301x; 1.0301x over previous
import functools

import jax
import jax.numpy as jnp
from jax import lax
from jax.experimental import pallas as pl
from jax.experimental.pallas import tpu as pltpu
from jax.experimental.pallas import tpu_sc as plsc

NN = 10000
NE = 160000
D = 128
H = 8
CQ = 16
CK = 16
TEMP = 4.0

EBLK = 2000
NBLK = 2000

SC_NC = 2
SC_NS = 16
SC_NW = SC_NC * SC_NS

BF = jnp.bfloat16
F32 = jnp.float32


def _bdot(a, b):
    return jnp.dot(a.astype(BF), b, preferred_element_type=F32)


def _gru_tail(gi, gh, h):
    r = jax.nn.sigmoid(gi[:, :D] + gh[:, :D])
    z = jax.nn.sigmoid(gi[:, D:2 * D] + gh[:, D:2 * D])
    n = jnp.tanh(gi[:, 2 * D:] + r * gh[:, 2 * D:])
    return (1.0 - z) * n + z * h


def _gru0_body(x_ref, wih_ref, bih_ref, bhh_ref, out_ref):
    gi = _bdot(x_ref[...], wih_ref[...]) + bih_ref[...]
    out_ref[...] = _gru_tail(gi, bhh_ref[...], 0.0)


def _gru0(x, wih_t, bih, bhh):
    nb = x.shape[0] // NBLK
    return pl.pallas_call(
        _gru0_body,
        grid=(nb,),
        in_specs=[
            pl.BlockSpec((NBLK, D), lambda i: (i, 0)),
            pl.BlockSpec((D, 3 * D), lambda i: (0, 0)),
            pl.BlockSpec((1, 3 * D), lambda i: (0, 0)),
            pl.BlockSpec((1, 3 * D), lambda i: (0, 0)),
        ],
        out_specs=pl.BlockSpec((NBLK, D), lambda i: (i, 0)),
        out_shape=jax.ShapeDtypeStruct((x.shape[0], D), F32),
    )(x, wih_t, bih, bhh)


def _edge_body(first,
               ea_ref, xi_ref, xj_ref,
               wih_ref, bih_ref, whh_ref, bhh_ref,
               w1a_ref, w1b_ref, w1c_ref, be1_ref, w2_ref, be2_ref,
               wq_ref, bq_ref, wk_ref, bk_ref, wv_ref, bv_ref,
               m1_ref, a1_ref, m2_ref, a2_ref, g_ref, gt_ref,
               enext_ref, prob_ref, value_ref):
    xi = xi_ref[...].astype(BF)
    xj = xj_ref[...].astype(BF)

    if first:
        gi = _bdot(ea_ref[...], wih_ref[...]) + bih_ref[...]
        e32 = _gru_tail(gi, bhh_ref[...], 0.0)
        eb = e32.astype(BF)
    else:
        eb = ea_ref[...]
        e32 = eb.astype(F32)

    h1 = jax.nn.relu(
        jnp.dot(xi, w1a_ref[...], preferred_element_type=F32)
        + jnp.dot(eb, w1b_ref[...], preferred_element_type=F32)
        + jnp.dot(xj, w1c_ref[...], preferred_element_type=F32)
        + be1_ref[...])
    emsg = _bdot(h1, w2_ref[...]) + be2_ref[...]

    q = jnp.dot(xi, wq_ref[...], preferred_element_type=F32) + bq_ref[...]
    k = jnp.dot(eb, wk_ref[...], preferred_element_type=F32) + bk_ref[...]
    v = jnp.dot(xj, wv_ref[...], preferred_element_type=F32) + bv_ref[...]
    ain = jnp.concatenate([q, k], axis=1)
    hcn = jax.nn.relu(_bdot(ain, m1_ref[...]) + a1_ref[...])
    att = (_bdot(hcn, m2_ref[...]) + a2_ref[...]) * (1.0 / TEMP)

    m = jnp.max(att, axis=1, keepdims=True)
    ex = jnp.exp(att - m)
    ssum = jnp.dot(ex, g_ref[...], preferred_element_type=F32)
    denom = jnp.dot(ssum, gt_ref[...], preferred_element_type=F32)
    prob = ex / denom
    prob_ref[...] = prob
    value_ref[...] = prob * v

    msg = jax.nn.relu(emsg) if first else emsg
    gi2 = _bdot(msg, wih_ref[...]) + bih_ref[...]
    gh2 = jnp.dot(eb, whh_ref[...], preferred_element_type=F32) + bhh_ref[...]
    enext = _gru_tail(gi2, gh2, e32)
    enext_ref[...] = enext.astype(enext_ref.dtype)


def _edge_layer(first, ea, xi, xj, ge, lp):
    nb = NE // EBLK
    blk = lambda r, c: pl.BlockSpec((r, c), lambda i: (0, 0))
    outs = pl.pallas_call(
        functools.partial(_edge_body, first),
        grid=(nb,),
        in_specs=[
            pl.BlockSpec((EBLK, D), lambda i: (i, 0)),
            pl.BlockSpec((EBLK, D), lambda i: (i, 0)),
            pl.BlockSpec((EBLK, D), lambda i: (i, 0)),
            blk(D, 3 * D), blk(1, 3 * D), blk(D, 3 * D), blk(1, 3 * D),
            blk(D, 2 * D), blk(D, 2 * D), blk(D, 2 * D), blk(1, 2 * D),
            blk(2 * D, D), blk(1, D),
            blk(D, D), blk(1, D), blk(D, D), blk(1, D), blk(D, D), blk(1, D),
            blk(2 * D, 2 * D), blk(1, 2 * D), blk(2 * D, D), blk(1, D),
            blk(D, H), blk(H, D),
        ],
        out_specs=[
            pl.BlockSpec((EBLK, D), lambda i: (i, 0)),
            pl.BlockSpec((EBLK, D), lambda i: (i, 0)),
            pl.BlockSpec((EBLK, D), lambda i: (i, 0)),
        ],
        out_shape=[
            jax.ShapeDtypeStruct((NE, D), BF if first else F32),
            jax.ShapeDtypeStruct((NE, D), F32),
            jax.ShapeDtypeStruct((NE, D), F32),
        ],
    )(ea, xi, xj,
      ge['wih'], ge['bih'], ge['whh'], ge['bhh'],
      lp['w1a'], lp['w1b'], lp['w1c'], lp['be1'], lp['w2'], lp['be2'],
      lp['wq'], lp['bq'], lp['wk'], lp['bk'], lp['wv'], lp['bv'],
      lp['m1'], lp['a1'], lp['m2'], lp['a2'], lp['g'], lp['gt'])
    return outs


def _node_body(first,
               node_ref, parts_ref,
               u1a_ref, u1b_ref, u1_ref, u2_ref, u2b_ref,
               wih_ref, bih_ref, whh_ref, bhh_ref,
               out_ref):
    node = node_ref[...]
    agg = parts_ref[0] + parts_ref[1]
    u = jax.nn.relu(
        _bdot(node, u1a_ref[...])
        + _bdot(agg, u1b_ref[...])
        + u1_ref[...])
    nmsg = _bdot(u, u2_ref[...]) + u2b_ref[...]
    if first:
        nmsg = jax.nn.relu(nmsg)
    gi = _bdot(nmsg, wih_ref[...]) + bih_ref[...]
    gh = _bdot(node, whh_ref[...]) + bhh_ref[...]
    out_ref[...] = _gru_tail(gi, gh, node)


def _node_layer(first, node, parts, gn, lp):
    nb = NN // NBLK
    blk = lambda r, c: pl.BlockSpec((r, c), lambda i: (0, 0))
    return pl.pallas_call(
        functools.partial(_node_body, first),
        grid=(nb,),
        in_specs=[
            pl.BlockSpec((NBLK, D), lambda i: (i, 0)),
            pl.BlockSpec((SC_NC, NBLK, D), lambda i: (0, i, 0)),
            blk(D, 2 * D), blk(D, 2 * D), blk(1, 2 * D),
            blk(2 * D, D), blk(1, D),
            blk(D, 3 * D), blk(1, 3 * D), blk(D, 3 * D), blk(1, 3 * D),
        ],
        out_specs=pl.BlockSpec((NBLK, D), lambda i: (i, 0)),
        out_shape=jax.ShapeDtypeStruct((NN, D), F32),
    )(node, parts,
      lp['u1a'], lp['u1b'], lp['u1'], lp['u2'], lp['u2b'],
      gn['wih'], gn['bih'], gn['whh'], gn['bhh'])


G_ROWS = 2 * NE
G_PER_W = G_ROWS // SC_NW
G_CHUNK = 400
G_ITERS = G_PER_W // G_CHUNK


def _gather_body(table, idx, out,
                 idx_v0, rows_v0, idx_v1, rows_v1,
                 gsem0, gsem1, wsem0, wsem1):
    c = lax.axis_index("c")
    s = lax.axis_index("s")
    base = (s * SC_NC + c) * G_PER_W
    bufs = [(idx_v0, rows_v0, gsem0, wsem0), (idx_v1, rows_v1, gsem1, wsem1)]
    writes = [None, None]
    for i in range(G_ITERS):
        idx_v, rows_v, gsem, wsem = bufs[i % 2]
        if writes[i % 2] is not None:
            writes[i % 2].wait()
        off = base + i * G_CHUNK
        pltpu.sync_copy(idx.at[pl.ds(off, G_CHUNK)], idx_v)
        pltpu.async_copy(table.at[idx_v], rows_v, gsem).wait()
        writes[i % 2] = pltpu.async_copy(rows_v, out.at[pl.ds(off, G_CHUNK)],
                                         wsem)
    for w in writes:
        if w is not None:
            w.wait()


def _sc_gather(table, idx_cat):
    call = pl.kernel(
        _gather_body,
        out_type=jax.ShapeDtypeStruct((G_ROWS, D), F32),
        mesh=plsc.VectorSubcoreMesh(core_axis_name="c", subcore_axis_name="s"),
        scratch_types=[
            pltpu.VMEM((G_CHUNK,), jnp.int32),
            pltpu.VMEM((G_CHUNK, D), F32),
            pltpu.VMEM((G_CHUNK,), jnp.int32),
            pltpu.VMEM((G_CHUNK, D), F32),
            pltpu.SemaphoreType.DMA,
            pltpu.SemaphoreType.DMA,
            pltpu.SemaphoreType.DMA,
            pltpu.SemaphoreType.DMA,
        ],
    )
    return call(table, idx_cat)


S_PER_SC = NE // SC_NC
S_PER_T = S_PER_SC // SC_NS
S_CHUNK = 200
S_ITERS = S_PER_T // S_CHUNK
NN_PAD = 10112
Z_ROWS = NN_PAD // SC_NS


def _scatter_body(value, idx, zrows, out, agg_sp, idx_v, val_v, lsem):
    c = lax.axis_index("c")
    s = lax.axis_index("s")
    pltpu.sync_copy(zrows, agg_sp.at[pl.ds(s * Z_ROWS, Z_ROWS)])
    plsc.subcore_barrier()
    base = c * S_PER_SC + s * S_PER_T
    for i in range(S_ITERS):
        off = base + i * S_CHUNK
        h1 = pltpu.async_copy(idx.at[pl.ds(off, S_CHUNK)], idx_v, lsem)
        h2 = pltpu.async_copy(value.at[pl.ds(off, S_CHUNK)], val_v, lsem)
        h1.wait()
        h2.wait()
        pltpu.sync_copy(val_v, agg_sp.at[idx_v], add=True)
    plsc.subcore_barrier()
    pltpu.sync_copy(agg_sp.at[pl.ds(s * Z_ROWS, Z_ROWS)],
                    out.at[c, pl.ds(s * Z_ROWS, Z_ROWS)])


def _sc_scatter(value, idx_i):
    call = pl.kernel(
        _scatter_body,
        out_type=jax.ShapeDtypeStruct((SC_NC, NN_PAD, D), F32),
        mesh=plsc.VectorSubcoreMesh(core_axis_name="c", subcore_axis_name="s"),
        scratch_types=[
            pltpu.VMEM_SHARED((NN_PAD, D), F32),
            pltpu.VMEM((S_CHUNK,), jnp.int32),
            pltpu.VMEM((S_CHUNK, D), F32),
            pltpu.SemaphoreType.DMA,
        ],
    )
    zrows = jnp.zeros((Z_ROWS, D), F32)
    return call(value, idx_i, zrows)


def _prep_gru(p):
    return {
        'wih': p['Wih'].T.astype(BF),
        'whh': p['Whh'].T.astype(BF),
        'bih': p['bih'].reshape(1, -1),
        'bhh': p['bhh'].reshape(1, -1),
    }


def _prep_layer(p):
    eye = jnp.eye(H, dtype=F32)
    m1 = (p['A1'].T[:, None, :, None] * eye[None, :, None, :]).reshape(2 * D, 2 * D)
    m2 = (p['A2'].T[:, None, :, None] * eye[None, :, None, :]).reshape(2 * D, D)
    w1 = p['We1'].T
    g = jnp.tile(eye, (CK, 1))
    return {
        'w1a': w1[:D].astype(BF), 'w1b': w1[D:2 * D].astype(BF),
        'w1c': w1[2 * D:].astype(BF),
        'be1': p['be1'].reshape(1, -1),
        'w2': p['We2'].T.astype(BF), 'be2': p['be2'].reshape(1, -1),
        'wq': p['Wq'].T.astype(BF), 'bq': p['bq'].reshape(1, -1),
        'wk': p['Wk'].T.astype(BF), 'bk': p['bk'].reshape(1, -1),
        'wv': p['Wv'].T.astype(BF), 'bv': p['bv'].reshape(1, -1),
        'm1': m1.astype(BF), 'a1': jnp.repeat(p['a1'], H).reshape(1, -1),
        'm2': m2.astype(BF), 'a2': jnp.repeat(p['a2'], H).reshape(1, -1),
        'g': g, 'gt': g.T,
        'u1a': p['U1'].T[:D].astype(BF), 'u1b': p['U1'].T[D:].astype(BF),
        'u1': p['u1'].reshape(1, -1),
        'u2': p['U2'].T.astype(BF), 'u2b': p['u2'].reshape(1, -1),
    }


def kernel(x, edge_attr, edge_index, params):
    idx_i = edge_index[0]
    idx_cat = jnp.concatenate([edge_index[0], edge_index[1]])
    gn = _prep_gru(params['gru_node'])
    ge = _prep_gru(params['gru_edge'])

    node = _gru0(x, gn['wih'], gn['bih'], gn['bhh'])
    edge = edge_attr
    probs = []
    for li in range(2):
        lp = _prep_layer(params['layers'][li])
        first = li == 0
        gath = _sc_gather(node, idx_cat)
        xi = gath[:NE]
        xj = gath[NE:]
        edge, prob, value = _edge_layer(first, edge, xi, xj, ge, lp)
        parts = _sc_scatter(value, idx_i)
        node = _node_layer(first, node, parts, gn, lp)
        probs.append(prob)
    return node, edge, jnp.stack(probs).reshape(2, NE, CK, H)


# speedup vs baseline: 1.2262x; 1.1905x over previous
import functools

import jax
import jax.numpy as jnp
from jax import lax
from jax.experimental import pallas as pl
from jax.experimental.pallas import tpu as pltpu
from jax.experimental.pallas import tpu_sc as plsc

NN = 10000
NE = 160000
D = 128
H = 8
CQ = 16
CK = 16
TEMP = 4.0

EBLK = 4000
NBLK = 2000

SC_NC = 2
SC_NS = 16
SC_NW = SC_NC * SC_NS

BF = jnp.bfloat16
F32 = jnp.float32


def _bdot(a, b):
    return jnp.dot(a.astype(BF), b, preferred_element_type=F32)


def _gru_tail(gi, gh, h):
    r = jax.nn.sigmoid(gi[:, :D] + gh[:, :D])
    z = jax.nn.sigmoid(gi[:, D:2 * D] + gh[:, D:2 * D])
    n = jnp.tanh(gi[:, 2 * D:] + r * gh[:, 2 * D:])
    return (1.0 - z) * n + z * h


def _gru0_body(x_ref, wih_ref, bih_ref, bhh_ref, out_ref):
    gi = _bdot(x_ref[...], wih_ref[...]) + bih_ref[...]
    out_ref[...] = _gru_tail(gi, bhh_ref[...], 0.0)


def _gru0(x, wih_t, bih, bhh):
    nb = x.shape[0] // NBLK
    return pl.pallas_call(
        _gru0_body,
        grid=(nb,),
        in_specs=[
            pl.BlockSpec((NBLK, D), lambda i: (i, 0)),
            pl.BlockSpec((D, 3 * D), lambda i: (0, 0)),
            pl.BlockSpec((1, 3 * D), lambda i: (0, 0)),
            pl.BlockSpec((1, 3 * D), lambda i: (0, 0)),
        ],
        out_specs=pl.BlockSpec((NBLK, D), lambda i: (i, 0)),
        out_shape=jax.ShapeDtypeStruct((x.shape[0], D), F32),
    )(x, wih_t, bih, bhh)


def _edge_body(first,
               ea_ref, xi_ref, xj_ref,
               wih_ref, bih_ref, whh_ref, bhh_ref,
               w1a_ref, w1b_ref, w1c_ref, be1_ref, w2_ref, be2_ref,
               wq_ref, bq_ref, wk_ref, bk_ref, wv_ref, bv_ref,
               m1_ref, a1_ref, m2_ref, a2_ref, g_ref, gt_ref,
               enext_ref, prob_ref, value_ref):
    xi = xi_ref[...].astype(BF)
    xj = xj_ref[...].astype(BF)

    if first:
        gi = _bdot(ea_ref[...], wih_ref[...]) + bih_ref[...]
        e32 = _gru_tail(gi, bhh_ref[...], 0.0)
        eb = e32.astype(BF)
    else:
        eb = ea_ref[...]
        e32 = eb.astype(F32)

    h1 = jax.nn.relu(
        jnp.dot(xi, w1a_ref[...], preferred_element_type=F32)
        + jnp.dot(eb, w1b_ref[...], preferred_element_type=F32)
        + jnp.dot(xj, w1c_ref[...], preferred_element_type=F32)
        + be1_ref[...])
    emsg = _bdot(h1, w2_ref[...]) + be2_ref[...]

    q = jnp.dot(xi, wq_ref[...], preferred_element_type=F32) + bq_ref[...]
    k = jnp.dot(eb, wk_ref[...], preferred_element_type=F32) + bk_ref[...]
    v = jnp.dot(xj, wv_ref[...], preferred_element_type=F32) + bv_ref[...]
    ain = jnp.concatenate([q, k], axis=1)
    hcn = jax.nn.relu(_bdot(ain, m1_ref[...]) + a1_ref[...])
    att = (_bdot(hcn, m2_ref[...]) + a2_ref[...]) * (1.0 / TEMP)

    m = jnp.max(att, axis=1, keepdims=True)
    ex = jnp.exp(att - m)
    ssum = jnp.dot(ex, g_ref[...], preferred_element_type=F32)
    denom = jnp.dot(ssum, gt_ref[...], preferred_element_type=F32)
    prob = ex / denom
    prob_ref[...] = prob
    value_ref[...] = prob * v

    msg = jax.nn.relu(emsg) if first else emsg
    gi2 = _bdot(msg, wih_ref[...]) + bih_ref[...]
    gh2 = jnp.dot(eb, whh_ref[...], preferred_element_type=F32) + bhh_ref[...]
    enext = _gru_tail(gi2, gh2, e32)
    enext_ref[...] = enext.astype(enext_ref.dtype)


def _edge_layer(first, ea, xi, xj, ge, lp):
    nb = NE // EBLK
    blk = lambda r, c: pl.BlockSpec((r, c), lambda i: (0, 0))
    outs = pl.pallas_call(
        functools.partial(_edge_body, first),
        grid=(nb,),
        in_specs=[
            pl.BlockSpec((EBLK, D), lambda i: (i, 0)),
            pl.BlockSpec((EBLK, D), lambda i: (i, 0)),
            pl.BlockSpec((EBLK, D), lambda i: (NE // EBLK + i, 0)),
            blk(D, 3 * D), blk(1, 3 * D), blk(D, 3 * D), blk(1, 3 * D),
            blk(D, 2 * D), blk(D, 2 * D), blk(D, 2 * D), blk(1, 2 * D),
            blk(2 * D, D), blk(1, D),
            blk(D, D), blk(1, D), blk(D, D), blk(1, D), blk(D, D), blk(1, D),
            blk(2 * D, 2 * D), blk(1, 2 * D), blk(2 * D, D), blk(1, D),
            blk(D, H), blk(H, D),
        ],
        out_specs=[
            pl.BlockSpec((EBLK, D), lambda i: (i, 0)),
            pl.BlockSpec((EBLK, D), lambda i: (i, 0)),
            pl.BlockSpec((EBLK, D), lambda i: (i, 0)),
        ],
        out_shape=[
            jax.ShapeDtypeStruct((NE, D), BF if first else F32),
            jax.ShapeDtypeStruct((NE, D), F32),
            jax.ShapeDtypeStruct((NE, D), F32),
        ],
    )(ea, xi, xj,
      ge['wih'], ge['bih'], ge['whh'], ge['bhh'],
      lp['w1a'], lp['w1b'], lp['w1c'], lp['be1'], lp['w2'], lp['be2'],
      lp['wq'], lp['bq'], lp['wk'], lp['bk'], lp['wv'], lp['bv'],
      lp['m1'], lp['a1'], lp['m2'], lp['a2'], lp['g'], lp['gt'])
    return outs


def _node_body(first,
               node_ref, parts_ref,
               u1a_ref, u1b_ref, u1_ref, u2_ref, u2b_ref,
               wih_ref, bih_ref, whh_ref, bhh_ref,
               out_ref):
    node = node_ref[...]
    agg = parts_ref[0] + parts_ref[1]
    u = jax.nn.relu(
        _bdot(node, u1a_ref[...])
        + _bdot(agg, u1b_ref[...])
        + u1_ref[...])
    nmsg = _bdot(u, u2_ref[...]) + u2b_ref[...]
    if first:
        nmsg = jax.nn.relu(nmsg)
    gi = _bdot(nmsg, wih_ref[...]) + bih_ref[...]
    gh = _bdot(node, whh_ref[...]) + bhh_ref[...]
    out_ref[...] = _gru_tail(gi, gh, node)


def _node_layer(first, node, parts, gn, lp):
    nb = NN // NBLK
    blk = lambda r, c: pl.BlockSpec((r, c), lambda i: (0, 0))
    return pl.pallas_call(
        functools.partial(_node_body, first),
        grid=(nb,),
        in_specs=[
            pl.BlockSpec((NBLK, D), lambda i: (i, 0)),
            pl.BlockSpec((SC_NC, NBLK, D), lambda i: (0, i, 0)),
            blk(D, 2 * D), blk(D, 2 * D), blk(1, 2 * D),
            blk(2 * D, D), blk(1, D),
            blk(D, 3 * D), blk(1, 3 * D), blk(D, 3 * D), blk(1, 3 * D),
        ],
        out_specs=pl.BlockSpec((NBLK, D), lambda i: (i, 0)),
        out_shape=jax.ShapeDtypeStruct((NN, D), F32),
    )(node, parts,
      lp['u1a'], lp['u1b'], lp['u1'], lp['u2'], lp['u2b'],
      gn['wih'], gn['bih'], gn['whh'], gn['bhh'])


G_ROWS = 2 * NE
G_PER_W = G_ROWS // SC_NW
G_CHUNK = 400
G_ITERS = G_PER_W // G_CHUNK


def _gather_body(table, idx, out,
                 idx_v0, rows_v0, idx_v1, rows_v1,
                 gsem0, gsem1, wsem0, wsem1):
    c = lax.axis_index("c")
    s = lax.axis_index("s")
    base = (s * SC_NC + c) * G_PER_W
    bufs = [(idx_v0, rows_v0, gsem0, wsem0), (idx_v1, rows_v1, gsem1, wsem1)]
    writes = [None, None]
    for i in range(G_ITERS):
        idx_v, rows_v, gsem, wsem = bufs[i % 2]
        if writes[i % 2] is not None:
            writes[i % 2].wait()
        off = base + i * G_CHUNK
        pltpu.sync_copy(idx.at[pl.ds(off, G_CHUNK)], idx_v)
        pltpu.async_copy(table.at[idx_v], rows_v, gsem).wait()
        writes[i % 2] = pltpu.async_copy(rows_v, out.at[pl.ds(off, G_CHUNK)],
                                         wsem)
    for w in writes:
        if w is not None:
            w.wait()


def _sc_gather(table, idx_cat):
    call = pl.kernel(
        _gather_body,
        out_type=jax.ShapeDtypeStruct((G_ROWS, D), F32),
        mesh=plsc.VectorSubcoreMesh(core_axis_name="c", subcore_axis_name="s"),
        scratch_types=[
            pltpu.VMEM((G_CHUNK,), jnp.int32),
            pltpu.VMEM((G_CHUNK, D), F32),
            pltpu.VMEM((G_CHUNK,), jnp.int32),
            pltpu.VMEM((G_CHUNK, D), F32),
            pltpu.SemaphoreType.DMA,
            pltpu.SemaphoreType.DMA,
            pltpu.SemaphoreType.DMA,
            pltpu.SemaphoreType.DMA,
        ],
    )
    return call(table, idx_cat)


S_PER_SC = NE // SC_NC
S_PER_T = S_PER_SC // SC_NS
S_CHUNK = 200
S_ITERS = S_PER_T // S_CHUNK
NN_PAD = 10112
Z_ROWS = NN_PAD // SC_NS


def _scatter_body(value, idx, zrows, out, agg_sp, idx_v, val_v, lsem):
    c = lax.axis_index("c")
    s = lax.axis_index("s")
    pltpu.sync_copy(zrows, agg_sp.at[pl.ds(s * Z_ROWS, Z_ROWS)])
    plsc.subcore_barrier()
    base = c * S_PER_SC + s * S_PER_T
    for i in range(S_ITERS):
        off = base + i * S_CHUNK
        h1 = pltpu.async_copy(idx.at[pl.ds(off, S_CHUNK)], idx_v, lsem)
        h2 = pltpu.async_copy(value.at[pl.ds(off, S_CHUNK)], val_v, lsem)
        h1.wait()
        h2.wait()
        pltpu.sync_copy(val_v, agg_sp.at[idx_v], add=True)
    plsc.subcore_barrier()
    pltpu.sync_copy(agg_sp.at[pl.ds(s * Z_ROWS, Z_ROWS)],
                    out.at[c, pl.ds(s * Z_ROWS, Z_ROWS)])


def _sc_scatter(value, idx_i):
    call = pl.kernel(
        _scatter_body,
        out_type=jax.ShapeDtypeStruct((SC_NC, NN_PAD, D), F32),
        mesh=plsc.VectorSubcoreMesh(core_axis_name="c", subcore_axis_name="s"),
        scratch_types=[
            pltpu.VMEM_SHARED((NN_PAD, D), F32),
            pltpu.VMEM((S_CHUNK,), jnp.int32),
            pltpu.VMEM((S_CHUNK, D), F32),
            pltpu.SemaphoreType.DMA,
        ],
    )
    zrows = jnp.zeros((Z_ROWS, D), F32)
    return call(value, idx_i, zrows)


def _prep_gru(p):
    return {
        'wih': p['Wih'].T.astype(BF),
        'whh': p['Whh'].T.astype(BF),
        'bih': p['bih'].reshape(1, -1),
        'bhh': p['bhh'].reshape(1, -1),
    }


def _prep_layer(p):
    eye = jnp.eye(H, dtype=F32)
    m1 = (p['A1'].T[:, None, :, None] * eye[None, :, None, :]).reshape(2 * D, 2 * D)
    m2 = (p['A2'].T[:, None, :, None] * eye[None, :, None, :]).reshape(2 * D, D)
    w1 = p['We1'].T
    g = jnp.tile(eye, (CK, 1))
    return {
        'w1a': w1[:D].astype(BF), 'w1b': w1[D:2 * D].astype(BF),
        'w1c': w1[2 * D:].astype(BF),
        'be1': p['be1'].reshape(1, -1),
        'w2': p['We2'].T.astype(BF), 'be2': p['be2'].reshape(1, -1),
        'wq': p['Wq'].T.astype(BF), 'bq': p['bq'].reshape(1, -1),
        'wk': p['Wk'].T.astype(BF), 'bk': p['bk'].reshape(1, -1),
        'wv': p['Wv'].T.astype(BF), 'bv': p['bv'].reshape(1, -1),
        'm1': m1.astype(BF), 'a1': jnp.repeat(p['a1'], H).reshape(1, -1),
        'm2': m2.astype(BF), 'a2': jnp.repeat(p['a2'], H).reshape(1, -1),
        'g': g, 'gt': g.T,
        'u1a': p['U1'].T[:D].astype(BF), 'u1b': p['U1'].T[D:].astype(BF),
        'u1': p['u1'].reshape(1, -1),
        'u2': p['U2'].T.astype(BF), 'u2b': p['u2'].reshape(1, -1),
    }


def kernel(x, edge_attr, edge_index, params):
    idx_i = edge_index[0]
    idx_cat = jnp.concatenate([edge_index[0], edge_index[1]])
    gn = _prep_gru(params['gru_node'])
    ge = _prep_gru(params['gru_edge'])

    node = _gru0(x, gn['wih'], gn['bih'], gn['bhh'])
    edge = edge_attr
    probs = []
    for li in range(2):
        lp = _prep_layer(params['layers'][li])
        first = li == 0
        gath = _sc_gather(node, idx_cat)
        edge, prob, value = _edge_layer(first, edge, gath, gath, ge, lp)
        parts = _sc_scatter(value, idx_i)
        node = _node_layer(first, node, parts, gn, lp)
        probs.append(prob)
    return node, edge, jnp.stack(probs).reshape(2, NE, CK, H)


# speedup vs baseline: 1.3142x; 1.0718x over previous
import functools

import jax
import jax.numpy as jnp
from jax import lax
from jax.experimental import pallas as pl
from jax.experimental.pallas import tpu as pltpu
from jax.experimental.pallas import tpu_sc as plsc

NN = 10000
NE = 160000
D = 128
H = 8
CQ = 16
CK = 16
TEMP = 4.0

EBLK = 4000
NBLK = 2000

SC_NC = 2
SC_NS = 16
SC_NW = SC_NC * SC_NS

BF = jnp.bfloat16
F32 = jnp.float32


def _bdot(a, b):
    return jnp.dot(a.astype(BF), b, preferred_element_type=F32)


def _gru_tail(gi, gh, h):
    r = jax.nn.sigmoid(gi[:, :D] + gh[:, :D])
    z = jax.nn.sigmoid(gi[:, D:2 * D] + gh[:, D:2 * D])
    n = jnp.tanh(gi[:, 2 * D:] + r * gh[:, 2 * D:])
    return (1.0 - z) * n + z * h


def _gru0_body(x_ref, wih_ref, bih_ref, bhh_ref, out_ref):
    gi = _bdot(x_ref[...], wih_ref[...]) + bih_ref[...]
    out_ref[...] = _gru_tail(gi, bhh_ref[...], 0.0)


def _gru0(x, wih_t, bih, bhh):
    nb = x.shape[0] // NBLK
    return pl.pallas_call(
        _gru0_body,
        grid=(nb,),
        in_specs=[
            pl.BlockSpec((NBLK, D), lambda i: (i, 0)),
            pl.BlockSpec((D, 3 * D), lambda i: (0, 0)),
            pl.BlockSpec((1, 3 * D), lambda i: (0, 0)),
            pl.BlockSpec((1, 3 * D), lambda i: (0, 0)),
        ],
        out_specs=pl.BlockSpec((NBLK, D), lambda i: (i, 0)),
        out_shape=jax.ShapeDtypeStruct((x.shape[0], D), F32),
    )(x, wih_t, bih, bhh)


def _edge_body(first,
               ea_ref, xi_ref, xj_ref, *refs):
    if first:
        _edge_math(first, ea_ref, xi_ref, xj_ref, *refs)
    else:
        _edge_math(first, ea_ref, xi_ref, xj_ref, *refs[1:])


def _edge_math(first,
               ea_ref, xi_ref, xj_ref,
               wih_ref, bih_ref, whh_ref, bhh_ref,
               w1a_ref, w1b_ref, w1c_ref, be1_ref, w2_ref, be2_ref,
               wq_ref, bq_ref, wk_ref, bk_ref, wv_ref, bv_ref,
               m1_ref, a1_ref, m2_ref, a2_ref, g_ref, gt_ref,
               enext_ref, prob_ref, value_ref):
    xi = xi_ref[...].astype(BF)
    xj = xj_ref[...].astype(BF)

    if first:
        gi = _bdot(ea_ref[...], wih_ref[...]) + bih_ref[...]
        e32 = _gru_tail(gi, bhh_ref[...], 0.0)
        eb = e32.astype(BF)
    else:
        eb = ea_ref[...]
        e32 = eb.astype(F32)

    h1 = jax.nn.relu(
        jnp.dot(xi, w1a_ref[...], preferred_element_type=F32)
        + jnp.dot(eb, w1b_ref[...], preferred_element_type=F32)
        + jnp.dot(xj, w1c_ref[...], preferred_element_type=F32)
        + be1_ref[...])
    emsg = _bdot(h1, w2_ref[...]) + be2_ref[...]

    q = jnp.dot(xi, wq_ref[...], preferred_element_type=F32) + bq_ref[...]
    k = jnp.dot(eb, wk_ref[...], preferred_element_type=F32) + bk_ref[...]
    v = jnp.dot(xj, wv_ref[...], preferred_element_type=F32) + bv_ref[...]
    ain = jnp.concatenate([q, k], axis=1)
    hcn = jax.nn.relu(_bdot(ain, m1_ref[...]) + a1_ref[...])
    att = (_bdot(hcn, m2_ref[...]) + a2_ref[...]) * (1.0 / TEMP)

    m = jnp.max(att, axis=1, keepdims=True)
    ex = jnp.exp(att - m)
    ssum = jnp.dot(ex, g_ref[...], preferred_element_type=F32)
    denom = jnp.dot(ssum, gt_ref[...], preferred_element_type=F32)
    prob = ex / denom
    prob_ref[0] = prob
    value_ref[...] = prob * v

    msg = jax.nn.relu(emsg) if first else emsg
    gi2 = _bdot(msg, wih_ref[...]) + bih_ref[...]
    gh2 = jnp.dot(eb, whh_ref[...], preferred_element_type=F32) + bhh_ref[...]
    enext = _gru_tail(gi2, gh2, e32)
    enext_ref[...] = enext.astype(enext_ref.dtype)


def _edge_layer(first, ea, xi, xj, ge, lp, probbuf):
    nb = NE // EBLK
    li = 0 if first else 1
    blk = lambda r, c: pl.BlockSpec((r, c), lambda i: (0, 0))
    wspecs = [
        blk(D, 3 * D), blk(1, 3 * D), blk(D, 3 * D), blk(1, 3 * D),
        blk(D, 2 * D), blk(D, 2 * D), blk(D, 2 * D), blk(1, 2 * D),
        blk(2 * D, D), blk(1, D),
        blk(D, D), blk(1, D), blk(D, D), blk(1, D), blk(D, D), blk(1, D),
        blk(2 * D, 2 * D), blk(1, 2 * D), blk(2 * D, D), blk(1, D),
        blk(D, H), blk(H, D),
    ]
    in_specs = [
        pl.BlockSpec((EBLK, D), lambda i: (i, 0)),
        pl.BlockSpec((EBLK, D), lambda i: (i, 0)),
        pl.BlockSpec((EBLK, D), lambda i: (NE // EBLK + i, 0)),
    ]
    args = [ea, xi, xj]
    aliases = {}
    if not first:
        in_specs.append(pl.BlockSpec(memory_space=pl.ANY))
        args.append(probbuf)
        aliases = {3: 1}
    in_specs += wspecs
    args += [
        ge['wih'], ge['bih'], ge['whh'], ge['bhh'],
        lp['w1a'], lp['w1b'], lp['w1c'], lp['be1'], lp['w2'], lp['be2'],
        lp['wq'], lp['bq'], lp['wk'], lp['bk'], lp['wv'], lp['bv'],
        lp['m1'], lp['a1'], lp['m2'], lp['a2'], lp['g'], lp['gt'],
    ]
    outs = pl.pallas_call(
        functools.partial(_edge_body, first),
        grid=(nb,),
        in_specs=in_specs,
        out_specs=[
            pl.BlockSpec((EBLK, D), lambda i: (i, 0)),
            pl.BlockSpec((1, EBLK, D), lambda i, _li=li: (_li, i, 0)),
            pl.BlockSpec((EBLK, D), lambda i: (i, 0)),
        ],
        out_shape=[
            jax.ShapeDtypeStruct((NE, D), BF if first else F32),
            jax.ShapeDtypeStruct((2, NE, D), F32),
            jax.ShapeDtypeStruct((NE, D), F32),
        ],
        input_output_aliases=aliases,
    )(*args)
    return outs


def _node_body(first,
               node_ref, parts_ref,
               u1a_ref, u1b_ref, u1_ref, u2_ref, u2b_ref,
               wih_ref, bih_ref, whh_ref, bhh_ref,
               out_ref):
    node = node_ref[...]
    agg = parts_ref[0] + parts_ref[1]
    u = jax.nn.relu(
        _bdot(node, u1a_ref[...])
        + _bdot(agg, u1b_ref[...])
        + u1_ref[...])
    nmsg = _bdot(u, u2_ref[...]) + u2b_ref[...]
    if first:
        nmsg = jax.nn.relu(nmsg)
    gi = _bdot(nmsg, wih_ref[...]) + bih_ref[...]
    gh = _bdot(node, whh_ref[...]) + bhh_ref[...]
    out_ref[...] = _gru_tail(gi, gh, node)


def _node_layer(first, node, parts, gn, lp):
    nb = NN // NBLK
    blk = lambda r, c: pl.BlockSpec((r, c), lambda i: (0, 0))
    return pl.pallas_call(
        functools.partial(_node_body, first),
        grid=(nb,),
        in_specs=[
            pl.BlockSpec((NBLK, D), lambda i: (i, 0)),
            pl.BlockSpec((SC_NC, NBLK, D), lambda i: (0, i, 0)),
            blk(D, 2 * D), blk(D, 2 * D), blk(1, 2 * D),
            blk(2 * D, D), blk(1, D),
            blk(D, 3 * D), blk(1, 3 * D), blk(D, 3 * D), blk(1, 3 * D),
        ],
        out_specs=pl.BlockSpec((NBLK, D), lambda i: (i, 0)),
        out_shape=jax.ShapeDtypeStruct((NN, D), F32),
    )(node, parts,
      lp['u1a'], lp['u1b'], lp['u1'], lp['u2'], lp['u2b'],
      gn['wih'], gn['bih'], gn['whh'], gn['bhh'])


G_ROWS = 2 * NE
G_PER_W = G_ROWS // SC_NW
G_CHUNK = 400
G_ITERS = G_PER_W // G_CHUNK


def _gather_body(table, idx, out,
                 idx_v0, rows_v0, idx_v1, rows_v1,
                 gsem0, gsem1, wsem0, wsem1):
    c = lax.axis_index("c")
    s = lax.axis_index("s")
    base = (s * SC_NC + c) * G_PER_W
    bufs = [(idx_v0, rows_v0, gsem0, wsem0), (idx_v1, rows_v1, gsem1, wsem1)]
    writes = [None, None]
    for i in range(G_ITERS):
        idx_v, rows_v, gsem, wsem = bufs[i % 2]
        if writes[i % 2] is not None:
            writes[i % 2].wait()
        off = base + i * G_CHUNK
        pltpu.sync_copy(idx.at[pl.ds(off, G_CHUNK)], idx_v)
        pltpu.async_copy(table.at[idx_v], rows_v, gsem).wait()
        writes[i % 2] = pltpu.async_copy(rows_v, out.at[pl.ds(off, G_CHUNK)],
                                         wsem)
    for w in writes:
        if w is not None:
            w.wait()


def _sc_gather(table, idx_cat):
    call = pl.kernel(
        _gather_body,
        out_type=jax.ShapeDtypeStruct((G_ROWS, D), F32),
        mesh=plsc.VectorSubcoreMesh(core_axis_name="c", subcore_axis_name="s"),
        scratch_types=[
            pltpu.VMEM((G_CHUNK,), jnp.int32),
            pltpu.VMEM((G_CHUNK, D), F32),
            pltpu.VMEM((G_CHUNK,), jnp.int32),
            pltpu.VMEM((G_CHUNK, D), F32),
            pltpu.SemaphoreType.DMA,
            pltpu.SemaphoreType.DMA,
            pltpu.SemaphoreType.DMA,
            pltpu.SemaphoreType.DMA,
        ],
    )
    return call(table, idx_cat)


S_PER_SC = NE // SC_NC
S_PER_T = S_PER_SC // SC_NS
S_CHUNK = 200
S_ITERS = S_PER_T // S_CHUNK
NN_PAD = 10112
Z_ROWS = NN_PAD // SC_NS


def _scatter_body(value, idx, zrows, out, agg_sp, idx_v, val_v, lsem):
    c = lax.axis_index("c")
    s = lax.axis_index("s")
    pltpu.sync_copy(zrows, agg_sp.at[pl.ds(s * Z_ROWS, Z_ROWS)])
    plsc.subcore_barrier()
    base = c * S_PER_SC + s * S_PER_T
    for i in range(S_ITERS):
        off = base + i * S_CHUNK
        h1 = pltpu.async_copy(idx.at[pl.ds(off, S_CHUNK)], idx_v, lsem)
        h2 = pltpu.async_copy(value.at[pl.ds(off, S_CHUNK)], val_v, lsem)
        h1.wait()
        h2.wait()
        pltpu.sync_copy(val_v, agg_sp.at[idx_v], add=True)
    plsc.subcore_barrier()
    pltpu.sync_copy(agg_sp.at[pl.ds(s * Z_ROWS, Z_ROWS)],
                    out.at[c, pl.ds(s * Z_ROWS, Z_ROWS)])


def _sc_scatter(value, idx_i):
    call = pl.kernel(
        _scatter_body,
        out_type=jax.ShapeDtypeStruct((SC_NC, NN_PAD, D), F32),
        mesh=plsc.VectorSubcoreMesh(core_axis_name="c", subcore_axis_name="s"),
        scratch_types=[
            pltpu.VMEM_SHARED((NN_PAD, D), F32),
            pltpu.VMEM((S_CHUNK,), jnp.int32),
            pltpu.VMEM((S_CHUNK, D), F32),
            pltpu.SemaphoreType.DMA,
        ],
    )
    zrows = jnp.zeros((Z_ROWS, D), F32)
    return call(value, idx_i, zrows)


def _prep_gru(p):
    return {
        'wih': p['Wih'].T.astype(BF),
        'whh': p['Whh'].T.astype(BF),
        'bih': p['bih'].reshape(1, -1),
        'bhh': p['bhh'].reshape(1, -1),
    }


def _prep_layer(p):
    eye = jnp.eye(H, dtype=F32)
    m1 = (p['A1'].T[:, None, :, None] * eye[None, :, None, :]).reshape(2 * D, 2 * D)
    m2 = (p['A2'].T[:, None, :, None] * eye[None, :, None, :]).reshape(2 * D, D)
    w1 = p['We1'].T
    g = jnp.tile(eye, (CK, 1))
    return {
        'w1a': w1[:D].astype(BF), 'w1b': w1[D:2 * D].astype(BF),
        'w1c': w1[2 * D:].astype(BF),
        'be1': p['be1'].reshape(1, -1),
        'w2': p['We2'].T.astype(BF), 'be2': p['be2'].reshape(1, -1),
        'wq': p['Wq'].T.astype(BF), 'bq': p['bq'].reshape(1, -1),
        'wk': p['Wk'].T.astype(BF), 'bk': p['bk'].reshape(1, -1),
        'wv': p['Wv'].T.astype(BF), 'bv': p['bv'].reshape(1, -1),
        'm1': m1.astype(BF), 'a1': jnp.repeat(p['a1'], H).reshape(1, -1),
        'm2': m2.astype(BF), 'a2': jnp.repeat(p['a2'], H).reshape(1, -1),
        'g': g, 'gt': g.T,
        'u1a': p['U1'].T[:D].astype(BF), 'u1b': p['U1'].T[D:].astype(BF),
        'u1': p['u1'].reshape(1, -1),
        'u2': p['U2'].T.astype(BF), 'u2b': p['u2'].reshape(1, -1),
    }


def kernel(x, edge_attr, edge_index, params):
    idx_i = edge_index[0]
    idx_cat = jnp.concatenate([edge_index[0], edge_index[1]])
    gn = _prep_gru(params['gru_node'])
    ge = _prep_gru(params['gru_edge'])

    node = _gru0(x, gn['wih'], gn['bih'], gn['bhh'])
    edge = edge_attr
    probbuf = None
    for li in range(2):
        lp = _prep_layer(params['layers'][li])
        first = li == 0
        gath = _sc_gather(node, idx_cat)
        edge, probbuf, value = _edge_layer(first, edge, gath, gath, ge, lp,
                                           probbuf)
        parts = _sc_scatter(value, idx_i)
        node = _node_layer(first, node, parts, gn, lp)
    return node, edge, probbuf.reshape(2, NE, CK, H)


# speedup vs baseline: 1.4906x; 1.1342x over previous
import functools

import jax
import jax.numpy as jnp
from jax import lax
from jax.experimental import pallas as pl
from jax.experimental.pallas import tpu as pltpu
from jax.experimental.pallas import tpu_sc as plsc

NN = 10000
NE = 160000
D = 128
H = 8
CQ = 16
CK = 16
TEMP = 4.0

EBLK = 4000
NBLK = 2000

SC_NC = 2
SC_NS = 16
SC_NW = SC_NC * SC_NS

BF = jnp.bfloat16
F32 = jnp.float32


def _bdot(a, b, out=F32):
    r = jnp.dot(a.astype(BF), b, preferred_element_type=F32)
    return r.astype(out)


def _gru_tail(gi, gh, h):
    r = jax.nn.sigmoid(gi[:, :D] + gh[:, :D])
    z = jax.nn.sigmoid(gi[:, D:2 * D] + gh[:, D:2 * D])
    n = jnp.tanh(gi[:, 2 * D:] + r * gh[:, 2 * D:])
    return (1.0 - z) * n + z * h


def _gru0_body(x_ref, wih_ref, bih_ref, bhh_ref, out_ref):
    gi = _bdot(x_ref[...], wih_ref[...]) + bih_ref[...]
    out_ref[...] = _gru_tail(gi, bhh_ref[...], 0.0)


def _gru0(x, wih_t, bih, bhh):
    nb = x.shape[0] // NBLK
    return pl.pallas_call(
        _gru0_body,
        grid=(nb,),
        in_specs=[
            pl.BlockSpec((NBLK, D), lambda i: (i, 0)),
            pl.BlockSpec((D, 3 * D), lambda i: (0, 0)),
            pl.BlockSpec((1, 3 * D), lambda i: (0, 0)),
            pl.BlockSpec((1, 3 * D), lambda i: (0, 0)),
        ],
        out_specs=pl.BlockSpec((NBLK, D), lambda i: (i, 0)),
        out_shape=jax.ShapeDtypeStruct((x.shape[0], D), F32),
    )(x, wih_t, bih, bhh)


def _edge_body(first,
               ea_ref, xi_ref, xj_ref, *refs):
    if first:
        _edge_math(first, ea_ref, xi_ref, xj_ref, *refs)
    else:
        _edge_math(first, ea_ref, xi_ref, xj_ref, *refs[1:])


def _gru_gates(gi, gh):
    r = jax.nn.sigmoid(gi[:, :D] + gh[:, :D])
    z = jax.nn.sigmoid(gi[:, D:2 * D] + gh[:, D:2 * D])
    n = jnp.tanh(gi[:, 2 * D:] + r * gh[:, 2 * D:])
    return z.astype(F32), n.astype(F32)


def _edge_math(first,
               ea_ref, xi_ref, xj_ref,
               wih_ref, bih_ref, whh_ref, bhh_ref,
               w1_ref, be1_ref, w2_ref, be2_ref,
               wq_ref, bq_ref, wk_ref, bk_ref, wv_ref, bv_ref,
               m1_ref, a1_ref, m2_ref, a2_ref, g_ref, gt_ref,
               enext_ref, prob_ref, value_ref):
    xi = xi_ref[...].astype(BF)
    xj = xj_ref[...].astype(BF)

    if first:
        gi = _bdot(ea_ref[...], wih_ref[...], BF) + bih_ref[...]
        z0, n0 = _gru_gates(gi, jnp.broadcast_to(bhh_ref[...], gi.shape))
        e32 = (1.0 - z0) * n0
        eb = e32.astype(BF)
    else:
        eb = ea_ref[...]
        e32 = eb.astype(F32)

    trip = jnp.concatenate([xi, eb, xj], axis=1)
    h1 = jax.nn.relu(_bdot(trip, w1_ref[...], BF) + be1_ref[...])
    emsg = _bdot(h1, w2_ref[...], BF) + be2_ref[...]

    q = _bdot(xi, wq_ref[...], BF) + bq_ref[...]
    k = _bdot(eb, wk_ref[...], BF) + bk_ref[...]
    v = jnp.dot(xj, wv_ref[...], preferred_element_type=F32) + bv_ref[...]
    ain = jnp.concatenate([q, k], axis=1)
    hcn = jax.nn.relu(_bdot(ain, m1_ref[...], BF) + a1_ref[...])
    att = (_bdot(hcn, m2_ref[...]) + a2_ref[...]) * (1.0 / TEMP)

    m = jnp.max(att, axis=1, keepdims=True)
    ex = jnp.exp(att - m)
    ssum = jnp.dot(ex, g_ref[...], preferred_element_type=F32)
    denom = jnp.dot(ssum, gt_ref[...], preferred_element_type=F32)
    prob = ex / denom
    prob_ref[0] = prob
    value_ref[...] = prob * v

    msg = jax.nn.relu(emsg) if first else emsg
    gi2 = _bdot(msg, wih_ref[...], BF) + bih_ref[...]
    gh2 = _bdot(eb, whh_ref[...], BF) + bhh_ref[...]
    z2, n2 = _gru_gates(gi2, gh2)
    enext = (1.0 - z2) * n2 + z2 * e32
    enext_ref[...] = enext.astype(enext_ref.dtype)


def _edge_layer(first, ea, xi, xj, ge, lp, probbuf):
    nb = NE // EBLK
    li = 0 if first else 1
    blk = lambda r, c: pl.BlockSpec((r, c), lambda i: (0, 0))
    wspecs = [
        blk(D, 3 * D), blk(1, 3 * D), blk(D, 3 * D), blk(1, 3 * D),
        blk(3 * D, 2 * D), blk(1, 2 * D),
        blk(2 * D, D), blk(1, D),
        blk(D, D), blk(1, D), blk(D, D), blk(1, D), blk(D, D), blk(1, D),
        blk(2 * D, 2 * D), blk(1, 2 * D), blk(2 * D, D), blk(1, D),
        blk(D, H), blk(H, D),
    ]
    in_specs = [
        pl.BlockSpec((EBLK, D), lambda i: (i, 0)),
        pl.BlockSpec((EBLK, D), lambda i: (i, 0)),
        pl.BlockSpec((EBLK, D), lambda i: (NE // EBLK + i, 0)),
    ]
    args = [ea, xi, xj]
    aliases = {}
    if not first:
        in_specs.append(pl.BlockSpec(memory_space=pl.ANY))
        args.append(probbuf)
        aliases = {3: 1}
    in_specs += wspecs
    args += [
        ge['wih'], ge['bihb'], ge['whh'], ge['bhhb'],
        lp['w1'], lp['be1b'], lp['w2'], lp['be2b'],
        lp['wq'], lp['bqb'], lp['wk'], lp['bkb'], lp['wv'], lp['bv'],
        lp['m1'], lp['a1b'], lp['m2'], lp['a2'], lp['g'], lp['gt'],
    ]
    outs = pl.pallas_call(
        functools.partial(_edge_body, first),
        grid=(nb,),
        in_specs=in_specs,
        out_specs=[
            pl.BlockSpec((EBLK, D), lambda i: (i, 0)),
            pl.BlockSpec((1, EBLK, D), lambda i, _li=li: (_li, i, 0)),
            pl.BlockSpec((EBLK, D), lambda i: (i, 0)),
        ],
        out_shape=[
            jax.ShapeDtypeStruct((NE, D), BF if first else F32),
            jax.ShapeDtypeStruct((2, NE, D), F32),
            jax.ShapeDtypeStruct((NE, D), F32),
        ],
        input_output_aliases=aliases,
    )(*args)
    return outs


def _node_body(first,
               node_ref, parts_ref,
               u1a_ref, u1b_ref, u1_ref, u2_ref, u2b_ref,
               wih_ref, bih_ref, whh_ref, bhh_ref,
               out_ref):
    node = node_ref[...]
    agg = parts_ref[0] + parts_ref[1]
    u = jax.nn.relu(
        _bdot(node, u1a_ref[...])
        + _bdot(agg, u1b_ref[...])
        + u1_ref[...])
    nmsg = _bdot(u, u2_ref[...]) + u2b_ref[...]
    if first:
        nmsg = jax.nn.relu(nmsg)
    gi = _bdot(nmsg, wih_ref[...]) + bih_ref[...]
    gh = _bdot(node, whh_ref[...]) + bhh_ref[...]
    out_ref[...] = _gru_tail(gi, gh, node)


def _node_layer(first, node, parts, gn, lp):
    nb = NN // NBLK
    blk = lambda r, c: pl.BlockSpec((r, c), lambda i: (0, 0))
    return pl.pallas_call(
        functools.partial(_node_body, first),
        grid=(nb,),
        in_specs=[
            pl.BlockSpec((NBLK, D), lambda i: (i, 0)),
            pl.BlockSpec((SC_NC, NBLK, D), lambda i: (0, i, 0)),
            blk(D, 2 * D), blk(D, 2 * D), blk(1, 2 * D),
            blk(2 * D, D), blk(1, D),
            blk(D, 3 * D), blk(1, 3 * D), blk(D, 3 * D), blk(1, 3 * D),
        ],
        out_specs=pl.BlockSpec((NBLK, D), lambda i: (i, 0)),
        out_shape=jax.ShapeDtypeStruct((NN, D), F32),
    )(node, parts,
      lp['u1a'], lp['u1b'], lp['u1'], lp['u2'], lp['u2b'],
      gn['wih'], gn['bih'], gn['whh'], gn['bhh'])


G_ROWS = 2 * NE
G_PER_W = G_ROWS // SC_NW
G_CHUNK = 400
G_ITERS = G_PER_W // G_CHUNK


def _gather_body(table, idx, out,
                 idx_v0, rows_v0, idx_v1, rows_v1,
                 gsem0, gsem1, wsem0, wsem1, isem0, isem1):
    c = lax.axis_index("c")
    s = lax.axis_index("s")
    base = (s * SC_NC + c) * G_PER_W
    idxs = [idx_v0, idx_v1]
    rows = [rows_v0, rows_v1]
    gsems = [gsem0, gsem1]
    wsems = [wsem0, wsem1]
    isems = [isem0, isem1]

    def start_idx(i):
        return pltpu.async_copy(idx.at[pl.ds(base + i * G_CHUNK, G_CHUNK)],
                                idxs[i % 2], isems[i % 2])

    def start_gather(i):
        return pltpu.async_copy(table.at[idxs[i % 2]], rows[i % 2],
                                gsems[i % 2])

    def start_write(i):
        return pltpu.async_copy(rows[i % 2],
                                out.at[pl.ds(base + i * G_CHUNK, G_CHUNK)],
                                wsems[i % 2])

    start_idx(0).wait()
    g = start_gather(0)
    ih = start_idx(1)
    writes = [None, None]
    for i in range(G_ITERS):
        g.wait()
        if i + 1 < G_ITERS:
            ih.wait()
            if writes[(i + 1) % 2] is not None:
                writes[(i + 1) % 2].wait()
            g = start_gather(i + 1)
            if i + 2 < G_ITERS:
                ih = start_idx(i + 2)
        writes[i % 2] = start_write(i)
    for w in writes:
        if w is not None:
            w.wait()


def _sc_gather(table, idx_cat):
    call = pl.kernel(
        _gather_body,
        out_type=jax.ShapeDtypeStruct((G_ROWS, D), F32),
        mesh=plsc.VectorSubcoreMesh(core_axis_name="c", subcore_axis_name="s"),
        scratch_types=[
            pltpu.VMEM((G_CHUNK,), jnp.int32),
            pltpu.VMEM((G_CHUNK, D), F32),
            pltpu.VMEM((G_CHUNK,), jnp.int32),
            pltpu.VMEM((G_CHUNK, D), F32),
            pltpu.SemaphoreType.DMA,
            pltpu.SemaphoreType.DMA,
            pltpu.SemaphoreType.DMA,
            pltpu.SemaphoreType.DMA,
            pltpu.SemaphoreType.DMA,
            pltpu.SemaphoreType.DMA,
        ],
    )
    return call(table, idx_cat)


S_PER_SC = NE // SC_NC
S_PER_T = S_PER_SC // SC_NS
S_CHUNK = 200
S_ITERS = S_PER_T // S_CHUNK
NN_PAD = 10112
Z_ROWS = NN_PAD // SC_NS


def _scatter_body(value, idx, zrows, out, agg_sp, idx_v, val_v, lsem):
    c = lax.axis_index("c")
    s = lax.axis_index("s")
    pltpu.sync_copy(zrows, agg_sp.at[pl.ds(s * Z_ROWS, Z_ROWS)])
    plsc.subcore_barrier()
    base = c * S_PER_SC + s * S_PER_T
    for i in range(S_ITERS):
        off = base + i * S_CHUNK
        h1 = pltpu.async_copy(idx.at[pl.ds(off, S_CHUNK)], idx_v, lsem)
        h2 = pltpu.async_copy(value.at[pl.ds(off, S_CHUNK)], val_v, lsem)
        h1.wait()
        h2.wait()
        pltpu.sync_copy(val_v, agg_sp.at[idx_v], add=True)
    plsc.subcore_barrier()
    pltpu.sync_copy(agg_sp.at[pl.ds(s * Z_ROWS, Z_ROWS)],
                    out.at[c, pl.ds(s * Z_ROWS, Z_ROWS)])


def _sc_scatter(value, idx_i):
    call = pl.kernel(
        _scatter_body,
        out_type=jax.ShapeDtypeStruct((SC_NC, NN_PAD, D), F32),
        mesh=plsc.VectorSubcoreMesh(core_axis_name="c", subcore_axis_name="s"),
        scratch_types=[
            pltpu.VMEM_SHARED((NN_PAD, D), F32),
            pltpu.VMEM((S_CHUNK,), jnp.int32),
            pltpu.VMEM((S_CHUNK, D), F32),
            pltpu.SemaphoreType.DMA,
        ],
    )
    zrows = jnp.zeros((Z_ROWS, D), F32)
    return call(value, idx_i, zrows)


def _prep_gru(p):
    return {
        'wih': p['Wih'].T.astype(BF),
        'whh': p['Whh'].T.astype(BF),
        'bih': p['bih'].reshape(1, -1),
        'bhh': p['bhh'].reshape(1, -1),
        'bihb': p['bih'].reshape(1, -1).astype(BF),
        'bhhb': p['bhh'].reshape(1, -1).astype(BF),
    }


def _prep_layer(p):
    eye = jnp.eye(H, dtype=F32)
    m1 = (p['A1'].T[:, None, :, None] * eye[None, :, None, :]).reshape(2 * D, 2 * D)
    m2 = (p['A2'].T[:, None, :, None] * eye[None, :, None, :]).reshape(2 * D, D)
    w1 = p['We1'].T
    g = jnp.tile(eye, (CK, 1))
    return {
        'w1': w1.astype(BF),
        'be1b': p['be1'].reshape(1, -1).astype(BF),
        'w2': p['We2'].T.astype(BF),
        'be2b': p['be2'].reshape(1, -1).astype(BF),
        'wq': p['Wq'].T.astype(BF), 'bqb': p['bq'].reshape(1, -1).astype(BF),
        'wk': p['Wk'].T.astype(BF), 'bkb': p['bk'].reshape(1, -1).astype(BF),
        'wv': p['Wv'].T.astype(BF), 'bv': p['bv'].reshape(1, -1),
        'm1': m1.astype(BF),
        'a1b': jnp.repeat(p['a1'], H).reshape(1, -1).astype(BF),
        'm2': m2.astype(BF), 'a2': jnp.repeat(p['a2'], H).reshape(1, -1),
        'g': g, 'gt': g.T,
        'u1a': p['U1'].T[:D].astype(BF), 'u1b': p['U1'].T[D:].astype(BF),
        'u1': p['u1'].reshape(1, -1),
        'u2': p['U2'].T.astype(BF), 'u2b': p['u2'].reshape(1, -1),
    }


def kernel(x, edge_attr, edge_index, params):
    idx_i = edge_index[0]
    idx_cat = jnp.concatenate([edge_index[0], edge_index[1]])
    gn = _prep_gru(params['gru_node'])
    ge = _prep_gru(params['gru_edge'])

    node = _gru0(x, gn['wih'], gn['bih'], gn['bhh'])
    edge = edge_attr
    probbuf = None
    for li in range(2):
        lp = _prep_layer(params['layers'][li])
        first = li == 0
        gath = _sc_gather(node, idx_cat)
        edge, probbuf, value = _edge_layer(first, edge, gath, gath, ge, lp,
                                           probbuf)
        parts = _sc_scatter(value, idx_i)
        node = _node_layer(first, node, parts, gn, lp)
    return node, edge, probbuf.reshape(2, NE, CK, H)


# speedup vs baseline: 1.5207x; 1.0202x over previous
import functools

import jax
import jax.numpy as jnp
from jax import lax
from jax.experimental import pallas as pl
from jax.experimental.pallas import tpu as pltpu
from jax.experimental.pallas import tpu_sc as plsc

NN = 10000
NE = 160000
D = 128
H = 8
CQ = 16
CK = 16
TEMP = 4.0

EBLK = 4000
NBLK = 5000

SC_NC = 2
SC_NS = 16
SC_NW = SC_NC * SC_NS

BF = jnp.bfloat16
F32 = jnp.float32


def _bdot(a, b, out=F32):
    r = jnp.dot(a.astype(BF), b, preferred_element_type=F32)
    return r.astype(out)


def _gru_tail(gi, gh, h):
    r = jax.nn.sigmoid(gi[:, :D] + gh[:, :D])
    z = jax.nn.sigmoid(gi[:, D:2 * D] + gh[:, D:2 * D])
    n = jnp.tanh(gi[:, 2 * D:] + r * gh[:, 2 * D:])
    return (1.0 - z) * n + z * h


def _gru0_body(x_ref, wih_ref, bih_ref, bhh_ref, out_ref):
    gi = _bdot(x_ref[...], wih_ref[...]) + bih_ref[...]
    out_ref[...] = _gru_tail(gi, bhh_ref[...], 0.0)


def _gru0(x, wih_t, bih, bhh):
    nb = x.shape[0] // NBLK
    return pl.pallas_call(
        _gru0_body,
        grid=(nb,),
        in_specs=[
            pl.BlockSpec((NBLK, D), lambda i: (i, 0)),
            pl.BlockSpec((D, 3 * D), lambda i: (0, 0)),
            pl.BlockSpec((1, 3 * D), lambda i: (0, 0)),
            pl.BlockSpec((1, 3 * D), lambda i: (0, 0)),
        ],
        out_specs=pl.BlockSpec((NBLK, D), lambda i: (i, 0)),
        out_shape=jax.ShapeDtypeStruct((x.shape[0], D), F32),
    )(x, wih_t, bih, bhh)


def _edge_body(first,
               ea_ref, xi_ref, xj_ref, *refs):
    if first:
        _edge_math(first, ea_ref, xi_ref, xj_ref, *refs)
    else:
        _edge_math(first, ea_ref, xi_ref, xj_ref, *refs[1:])


def _gru_gates(gi, gh):
    r = jax.nn.sigmoid(gi[:, :D] + gh[:, :D])
    z = jax.nn.sigmoid(gi[:, D:2 * D] + gh[:, D:2 * D])
    n = jnp.tanh(gi[:, 2 * D:] + r * gh[:, 2 * D:])
    return z.astype(F32), n.astype(F32)


def _edge_math(first,
               ea_ref, xi_ref, xj_ref,
               wih_ref, bih_ref, whh_ref, bhh_ref,
               w1_ref, be1_ref, w2_ref, be2_ref,
               wq_ref, bq_ref, wk_ref, bk_ref, wv_ref, bv_ref,
               m1_ref, a1_ref, m2_ref, a2_ref, g_ref, gt_ref,
               enext_ref, prob_ref, value_ref):
    xi = xi_ref[...].astype(BF)
    xj = xj_ref[...].astype(BF)

    if first:
        gi = _bdot(ea_ref[...], wih_ref[...], BF) + bih_ref[...]
        z0, n0 = _gru_gates(gi, jnp.broadcast_to(bhh_ref[...], gi.shape))
        e32 = (1.0 - z0) * n0
        eb = e32.astype(BF)
    else:
        eb = ea_ref[...]
        e32 = eb.astype(F32)

    trip = jnp.concatenate([xi, eb, xj], axis=1)
    h1 = jax.nn.relu(_bdot(trip, w1_ref[...], BF) + be1_ref[...])
    emsg = _bdot(h1, w2_ref[...], BF) + be2_ref[...]

    q = _bdot(xi, wq_ref[...], BF) + bq_ref[...]
    k = _bdot(eb, wk_ref[...], BF) + bk_ref[...]
    v = jnp.dot(xj, wv_ref[...], preferred_element_type=F32) + bv_ref[...]
    ain = jnp.concatenate([q, k], axis=1)
    hcn = jax.nn.relu(_bdot(ain, m1_ref[...], BF) + a1_ref[...])
    att = (_bdot(hcn, m2_ref[...]) + a2_ref[...]) * (1.0 / TEMP)

    m = jnp.max(att, axis=1, keepdims=True)
    ex = jnp.exp(att - m)
    ssum = jnp.dot(ex, g_ref[...], preferred_element_type=F32)
    denom = jnp.dot(ssum, gt_ref[...], preferred_element_type=F32)
    prob = ex / denom
    prob_ref[0] = prob.astype(BF)
    value_ref[...] = prob * v

    msg = jax.nn.relu(emsg) if first else emsg
    gi2 = _bdot(msg, wih_ref[...], BF) + bih_ref[...]
    gh2 = _bdot(eb, whh_ref[...], BF) + bhh_ref[...]
    z2, n2 = _gru_gates(gi2, gh2)
    enext = (1.0 - z2) * n2 + z2 * e32
    enext_ref[...] = enext.astype(enext_ref.dtype)


def _edge_layer(first, ea, xi, xj, ge, lp, probbuf):
    nb = NE // EBLK
    li = 0 if first else 1
    blk = lambda r, c: pl.BlockSpec((r, c), lambda i: (0, 0))
    wspecs = [
        blk(D, 3 * D), blk(1, 3 * D), blk(D, 3 * D), blk(1, 3 * D),
        blk(3 * D, 2 * D), blk(1, 2 * D),
        blk(2 * D, D), blk(1, D),
        blk(D, D), blk(1, D), blk(D, D), blk(1, D), blk(D, D), blk(1, D),
        blk(2 * D, 2 * D), blk(1, 2 * D), blk(2 * D, D), blk(1, D),
        blk(D, H), blk(H, D),
    ]
    in_specs = [
        pl.BlockSpec((EBLK, D), lambda i: (i, 0)),
        pl.BlockSpec((EBLK, D), lambda i: (i, 0)),
        pl.BlockSpec((EBLK, D), lambda i: (NE // EBLK + i, 0)),
    ]
    args = [ea, xi, xj]
    aliases = {}
    if not first:
        in_specs.append(pl.BlockSpec(memory_space=pl.ANY))
        args.append(probbuf)
        aliases = {3: 1}
    in_specs += wspecs
    args += [
        ge['wih'], ge['bihb'], ge['whh'], ge['bhhb'],
        lp['w1'], lp['be1b'], lp['w2'], lp['be2b'],
        lp['wq'], lp['bqb'], lp['wk'], lp['bkb'], lp['wv'], lp['bv'],
        lp['m1'], lp['a1b'], lp['m2'], lp['a2'], lp['g'], lp['gt'],
    ]
    outs = pl.pallas_call(
        functools.partial(_edge_body, first),
        grid=(nb,),
        in_specs=in_specs,
        out_specs=[
            pl.BlockSpec((EBLK, D), lambda i: (i, 0)),
            pl.BlockSpec((1, EBLK, D), lambda i, _li=li: (_li, i, 0)),
            pl.BlockSpec((EBLK, D), lambda i: (i, 0)),
        ],
        out_shape=[
            jax.ShapeDtypeStruct((NE, D), BF if first else F32),
            jax.ShapeDtypeStruct((2, NE, D), BF),
            jax.ShapeDtypeStruct((NE, D), F32),
        ],
        input_output_aliases=aliases,
    )(*args)
    return outs


def _node_body(first,
               node_ref, parts_ref,
               u1a_ref, u1b_ref, u1_ref, u2_ref, u2b_ref,
               wih_ref, bih_ref, whh_ref, bhh_ref,
               out_ref):
    node = node_ref[...]
    agg = parts_ref[0] + parts_ref[1]
    u = jax.nn.relu(
        _bdot(node, u1a_ref[...])
        + _bdot(agg, u1b_ref[...])
        + u1_ref[...])
    nmsg = _bdot(u, u2_ref[...]) + u2b_ref[...]
    if first:
        nmsg = jax.nn.relu(nmsg)
    gi = _bdot(nmsg, wih_ref[...]) + bih_ref[...]
    gh = _bdot(node, whh_ref[...]) + bhh_ref[...]
    out_ref[...] = _gru_tail(gi, gh, node)


def _node_layer(first, node, parts, gn, lp):
    nb = NN // NBLK
    blk = lambda r, c: pl.BlockSpec((r, c), lambda i: (0, 0))
    return pl.pallas_call(
        functools.partial(_node_body, first),
        grid=(nb,),
        in_specs=[
            pl.BlockSpec((NBLK, D), lambda i: (i, 0)),
            pl.BlockSpec((SC_NC, NBLK, D), lambda i: (0, i, 0)),
            blk(D, 2 * D), blk(D, 2 * D), blk(1, 2 * D),
            blk(2 * D, D), blk(1, D),
            blk(D, 3 * D), blk(1, 3 * D), blk(D, 3 * D), blk(1, 3 * D),
        ],
        out_specs=pl.BlockSpec((NBLK, D), lambda i: (i, 0)),
        out_shape=jax.ShapeDtypeStruct((NN, D), F32),
    )(node, parts,
      lp['u1a'], lp['u1b'], lp['u1'], lp['u2'], lp['u2b'],
      gn['wih'], gn['bih'], gn['whh'], gn['bhh'])


G_ROWS = 2 * NE
G_PER_W = G_ROWS // SC_NW
G_CHUNK = 400
G_ITERS = G_PER_W // G_CHUNK


def _gather_body(table, idx, out,
                 idx_v0, rows_v0, idx_v1, rows_v1,
                 gsem0, gsem1, wsem0, wsem1, isem0, isem1):
    c = lax.axis_index("c")
    s = lax.axis_index("s")
    base = (s * SC_NC + c) * G_PER_W
    idxs = [idx_v0, idx_v1]
    rows = [rows_v0, rows_v1]
    gsems = [gsem0, gsem1]
    wsems = [wsem0, wsem1]
    isems = [isem0, isem1]

    def start_idx(i):
        return pltpu.async_copy(idx.at[pl.ds(base + i * G_CHUNK, G_CHUNK)],
                                idxs[i % 2], isems[i % 2])

    def start_gather(i):
        return pltpu.async_copy(table.at[idxs[i % 2]], rows[i % 2],
                                gsems[i % 2])

    def start_write(i):
        return pltpu.async_copy(rows[i % 2],
                                out.at[pl.ds(base + i * G_CHUNK, G_CHUNK)],
                                wsems[i % 2])

    start_idx(0).wait()
    g = start_gather(0)
    ih = start_idx(1)
    writes = [None, None]
    for i in range(G_ITERS):
        g.wait()
        if i + 1 < G_ITERS:
            ih.wait()
            if writes[(i + 1) % 2] is not None:
                writes[(i + 1) % 2].wait()
            g = start_gather(i + 1)
            if i + 2 < G_ITERS:
                ih = start_idx(i + 2)
        writes[i % 2] = start_write(i)
    for w in writes:
        if w is not None:
            w.wait()


def _sc_gather(table, idx_cat):
    call = pl.kernel(
        _gather_body,
        out_type=jax.ShapeDtypeStruct((G_ROWS, D), F32),
        mesh=plsc.VectorSubcoreMesh(core_axis_name="c", subcore_axis_name="s"),
        scratch_types=[
            pltpu.VMEM((G_CHUNK,), jnp.int32),
            pltpu.VMEM((G_CHUNK, D), F32),
            pltpu.VMEM((G_CHUNK,), jnp.int32),
            pltpu.VMEM((G_CHUNK, D), F32),
            pltpu.SemaphoreType.DMA,
            pltpu.SemaphoreType.DMA,
            pltpu.SemaphoreType.DMA,
            pltpu.SemaphoreType.DMA,
            pltpu.SemaphoreType.DMA,
            pltpu.SemaphoreType.DMA,
        ],
    )
    return call(table, idx_cat)


S_PER_SC = NE // SC_NC
S_PER_T = S_PER_SC // SC_NS
S_CHUNK = 200
S_ITERS = S_PER_T // S_CHUNK
NN_PAD = 10112
Z_ROWS = NN_PAD // SC_NS


def _scatter_body(value, idx, zrows, out, agg_sp, idx_v, val_v, lsem):
    c = lax.axis_index("c")
    s = lax.axis_index("s")
    pltpu.sync_copy(zrows, agg_sp.at[pl.ds(s * Z_ROWS, Z_ROWS)])
    plsc.subcore_barrier()
    base = c * S_PER_SC + s * S_PER_T
    for i in range(S_ITERS):
        off = base + i * S_CHUNK
        h1 = pltpu.async_copy(idx.at[pl.ds(off, S_CHUNK)], idx_v, lsem)
        h2 = pltpu.async_copy(value.at[pl.ds(off, S_CHUNK)], val_v, lsem)
        h1.wait()
        h2.wait()
        pltpu.sync_copy(val_v, agg_sp.at[idx_v], add=True)
    plsc.subcore_barrier()
    pltpu.sync_copy(agg_sp.at[pl.ds(s * Z_ROWS, Z_ROWS)],
                    out.at[c, pl.ds(s * Z_ROWS, Z_ROWS)])


def _sc_scatter(value, idx_i):
    call = pl.kernel(
        _scatter_body,
        out_type=jax.ShapeDtypeStruct((SC_NC, NN_PAD, D), F32),
        mesh=plsc.VectorSubcoreMesh(core_axis_name="c", subcore_axis_name="s"),
        scratch_types=[
            pltpu.VMEM_SHARED((NN_PAD, D), F32),
            pltpu.VMEM((S_CHUNK,), jnp.int32),
            pltpu.VMEM((S_CHUNK, D), F32),
            pltpu.SemaphoreType.DMA,
        ],
    )
    zrows = jnp.zeros((Z_ROWS, D), F32)
    return call(value, idx_i, zrows)


def _prep_gru(p):
    return {
        'wih': p['Wih'].T.astype(BF),
        'whh': p['Whh'].T.astype(BF),
        'bih': p['bih'].reshape(1, -1),
        'bhh': p['bhh'].reshape(1, -1),
        'bihb': p['bih'].reshape(1, -1).astype(BF),
        'bhhb': p['bhh'].reshape(1, -1).astype(BF),
    }


def _prep_layer(p):
    eye = jnp.eye(H, dtype=F32)
    m1 = (p['A1'].T[:, None, :, None] * eye[None, :, None, :]).reshape(2 * D, 2 * D)
    m2 = (p['A2'].T[:, None, :, None] * eye[None, :, None, :]).reshape(2 * D, D)
    w1 = p['We1'].T
    g = jnp.tile(eye, (CK, 1))
    return {
        'w1': w1.astype(BF),
        'be1b': p['be1'].reshape(1, -1).astype(BF),
        'w2': p['We2'].T.astype(BF),
        'be2b': p['be2'].reshape(1, -1).astype(BF),
        'wq': p['Wq'].T.astype(BF), 'bqb': p['bq'].reshape(1, -1).astype(BF),
        'wk': p['Wk'].T.astype(BF), 'bkb': p['bk'].reshape(1, -1).astype(BF),
        'wv': p['Wv'].T.astype(BF), 'bv': p['bv'].reshape(1, -1),
        'm1': m1.astype(BF),
        'a1b': jnp.repeat(p['a1'], H).reshape(1, -1).astype(BF),
        'm2': m2.astype(BF), 'a2': jnp.repeat(p['a2'], H).reshape(1, -1),
        'g': g, 'gt': g.T,
        'u1a': p['U1'].T[:D].astype(BF), 'u1b': p['U1'].T[D:].astype(BF),
        'u1': p['u1'].reshape(1, -1),
        'u2': p['U2'].T.astype(BF), 'u2b': p['u2'].reshape(1, -1),
    }


def kernel(x, edge_attr, edge_index, params):
    idx_i = edge_index[0]
    idx_cat = jnp.concatenate([edge_index[0], edge_index[1]])
    gn = _prep_gru(params['gru_node'])
    ge = _prep_gru(params['gru_edge'])

    node = _gru0(x, gn['wih'], gn['bih'], gn['bhh'])
    edge = edge_attr
    probbuf = None
    for li in range(2):
        lp = _prep_layer(params['layers'][li])
        first = li == 0
        gath = _sc_gather(node, idx_cat)
        edge, probbuf, value = _edge_layer(first, edge, gath, gath, ge, lp,
                                           probbuf)
        parts = _sc_scatter(value, idx_i)
        node = _node_layer(first, node, parts, gn, lp)
    return node, edge, probbuf.astype(F32).reshape(2, NE, CK, H)
